# Initial kernel scaffold; baseline (speedup 1.0000x reference)
import jax
import jax.numpy as jnp
from jax.experimental import pallas as pl


def kernel(x, adj, w1, b1, w2, b2):
    raise NotImplementedError("write your pallas kernel here")


# speedup vs baseline: 2.7093x; 2.7093x over previous
import functools

import jax
import jax.numpy as jnp
from jax.experimental import pallas as pl
from jax.experimental.pallas import tpu as pltpu


def _proj_kernel(x_ref, w_ref, o_ref):
    o_ref[...] = jnp.dot(
        x_ref[...].astype(jnp.bfloat16),
        w_ref[...],
        preferred_element_type=jnp.float32,
    ).astype(o_ref.dtype)


def _project(x, w_bf16, *, tm):
    n, f = x.shape
    h = w_bf16.shape[1]
    return pl.pallas_call(
        _proj_kernel,
        out_shape=jax.ShapeDtypeStruct((n, h), jnp.bfloat16),
        grid=(n // tm,),
        in_specs=[
            pl.BlockSpec((tm, f), lambda i: (i, 0)),
            pl.BlockSpec((f, h), lambda i: (0, 0)),
        ],
        out_specs=pl.BlockSpec((tm, h), lambda i: (i, 0)),
        compiler_params=pltpu.CompilerParams(
            dimension_semantics=("parallel",),
        ),
        cost_estimate=pl.CostEstimate(
            flops=2 * n * f * h,
            transcendentals=0,
            bytes_accessed=4 * n * f + 2 * f * h + 2 * n * h,
        ),
    )(x, w_bf16)


def _layer1_kernel(adj_ref, s1_ref, b1_ref, w2_ref, o_ref):
    acc = jnp.dot(
        adj_ref[...].astype(jnp.bfloat16),
        s1_ref[...],
        preferred_element_type=jnp.float32,
    )
    hid = jnp.maximum(acc + b1_ref[...], 0.0).astype(jnp.bfloat16)
    o_ref[...] = jnp.dot(
        hid, w2_ref[...], preferred_element_type=jnp.float32
    ).astype(o_ref.dtype)


def _layer1(adj, s1, b1_row, w2_bf16, *, tm):
    n = adj.shape[0]
    h = s1.shape[1]
    c = w2_bf16.shape[1]
    return pl.pallas_call(
        _layer1_kernel,
        out_shape=jax.ShapeDtypeStruct((n, c), jnp.bfloat16),
        grid=(n // tm,),
        in_specs=[
            pl.BlockSpec((tm, n), lambda i: (i, 0)),
            pl.BlockSpec((n, h), lambda i: (0, 0)),
            pl.BlockSpec((1, h), lambda i: (0, 0)),
            pl.BlockSpec((h, c), lambda i: (0, 0)),
        ],
        out_specs=pl.BlockSpec((tm, c), lambda i: (i, 0)),
        compiler_params=pltpu.CompilerParams(
            dimension_semantics=("parallel",),
            vmem_limit_bytes=52 * 1024 * 1024,
        ),
        cost_estimate=pl.CostEstimate(
            flops=2 * n * n * h + 2 * n * h * c,
            transcendentals=0,
            bytes_accessed=4 * n * n + 2 * n * h + 2 * n * c,
        ),
    )(adj, s1, b1_row, w2_bf16)


def _layer2_kernel(adj_ref, s2_ref, b2_ref, o_ref):
    acc = jnp.dot(
        adj_ref[...].astype(jnp.bfloat16),
        s2_ref[...],
        preferred_element_type=jnp.float32,
    )
    logits = acc + b2_ref[...]
    m = jnp.max(logits, axis=1, keepdims=True)
    shifted = logits - m
    lse = jnp.log(jnp.sum(jnp.exp(shifted), axis=1, keepdims=True))
    o_ref[...] = (shifted - lse).astype(o_ref.dtype)


def _layer2(adj, s2, b2_row, *, tm):
    n = adj.shape[0]
    c = s2.shape[1]
    return pl.pallas_call(
        _layer2_kernel,
        out_shape=jax.ShapeDtypeStruct((n, c), jnp.float32),
        grid=(n // tm,),
        in_specs=[
            pl.BlockSpec((tm, n), lambda i: (i, 0)),
            pl.BlockSpec((n, c), lambda i: (0, 0)),
            pl.BlockSpec((1, c), lambda i: (0, 0)),
        ],
        out_specs=pl.BlockSpec((tm, c), lambda i: (i, 0)),
        compiler_params=pltpu.CompilerParams(
            dimension_semantics=("parallel",),
            vmem_limit_bytes=52 * 1024 * 1024,
        ),
        cost_estimate=pl.CostEstimate(
            flops=2 * n * n * c,
            transcendentals=n * c,
            bytes_accessed=4 * n * n + 2 * n * c + 4 * n * c,
        ),
    )(adj, s2, b2_row)


def kernel(x, adj, w1, b1, w2, b2):
    n = x.shape[0]
    nhid = w1.shape[1]
    nclass = w2.shape[1]

    tm = 512 if n % 512 == 0 else 128

    w1b = w1.astype(jnp.bfloat16)
    w2b = w2.astype(jnp.bfloat16)
    b1r = b1.astype(jnp.float32).reshape(1, nhid)
    b2r = b2.astype(jnp.float32).reshape(1, nclass)

    s1 = _project(x, w1b, tm=tm)
    s2 = _layer1(adj, s1, b1r, w2b, tm=tm)
    return _layer2(adj, s2, b2r, tm=tm)


# speedup vs baseline: 3.3390x; 1.2325x over previous
import functools

import jax
import jax.numpy as jnp
from jax.experimental import pallas as pl
from jax.experimental.pallas import tpu as pltpu


def _gcn_kernel(x_ref, w1_ref, adj_ref, b1_ref, w2_ref, b2_ref, o_ref,
                s1_cache, adj_cache, s2_cache, *, ns, tm, cast_chunk):
    i = pl.program_id(0)
    n = adj_ref.shape[1]

    @pl.when(i < ns)
    def _phase0():
        r0 = pl.multiple_of(i * tm, tm)
        s1_cache[pl.ds(r0, tm), :] = jnp.dot(
            x_ref[...].astype(jnp.bfloat16), w1_ref[...],
            preferred_element_type=jnp.float32).astype(jnp.bfloat16)

    @pl.when((i >= ns) & (i < 2 * ns))
    def _phase1():
        r0 = pl.multiple_of((i - ns) * tm, tm)
        a_bf = adj_ref[...].astype(jnp.bfloat16)
        for c0 in range(0, n, cast_chunk):
            adj_cache[pl.ds(r0, tm), pl.ds(c0, cast_chunk)] = (
                a_bf[:, c0:c0 + cast_chunk])
        acc = jnp.dot(a_bf, s1_cache[...], preferred_element_type=jnp.float32)
        hid = jnp.maximum(acc + b1_ref[...], 0.0).astype(jnp.bfloat16)
        s2_cache[pl.ds(r0, tm), :] = jnp.dot(
            hid, w2_ref[...], preferred_element_type=jnp.float32
        ).astype(jnp.bfloat16)

    @pl.when(i >= 2 * ns)
    def _phase2():
        r0 = pl.multiple_of((i - 2 * ns) * tm, tm)
        a_bf = adj_cache[pl.ds(r0, tm), :]
        acc = jnp.dot(a_bf, s2_cache[...], preferred_element_type=jnp.float32)
        logits = acc + b2_ref[...]
        m = jnp.max(logits, axis=1, keepdims=True)
        shifted = logits - m
        lse = jnp.log(jnp.sum(jnp.exp(shifted), axis=1, keepdims=True))
        o_ref[...] = (shifted - lse).astype(o_ref.dtype)


def _gcn_call(x, adj, w1b, b1_row, w2b, b2_row, *, tm):
    n, f = x.shape
    h = w1b.shape[1]
    c = w2b.shape[1]
    ns = n // tm
    return pl.pallas_call(
        functools.partial(_gcn_kernel, ns=ns, tm=tm, cast_chunk=min(512, n)),
        out_shape=jax.ShapeDtypeStruct((n, c), jnp.float32),
        grid=(3 * ns,),
        in_specs=[
            pl.BlockSpec((tm, f), lambda i: (jnp.minimum(i, ns - 1), 0)),
            pl.BlockSpec((f, h), lambda i: (0, 0)),
            pl.BlockSpec((tm, n),
                         lambda i: (jnp.clip(i - ns, 0, ns - 1), 0)),
            pl.BlockSpec((1, h), lambda i: (0, 0)),
            pl.BlockSpec((h, c), lambda i: (0, 0)),
            pl.BlockSpec((1, c), lambda i: (0, 0)),
        ],
        out_specs=pl.BlockSpec((tm, c),
                               lambda i: (jnp.clip(i - 2 * ns, 0, ns - 1), 0)),
        scratch_shapes=[
            pltpu.VMEM((n, h), jnp.bfloat16),
            pltpu.VMEM((n, n), jnp.bfloat16),
            pltpu.VMEM((n, c), jnp.bfloat16),
        ],
        compiler_params=pltpu.CompilerParams(
            dimension_semantics=("arbitrary",),
            vmem_limit_bytes=56 * 1024 * 1024,
        ),
        cost_estimate=pl.CostEstimate(
            flops=2 * n * f * h + 2 * n * n * h + 2 * n * h * c + 2 * n * n * c,
            transcendentals=n * c,
            bytes_accessed=4 * n * f + 4 * n * n + 6 * n * c,
        ),
    )(x, w1b, adj, b1_row, w2b, b2_row)


def kernel(x, adj, w1, b1, w2, b2):
    n = x.shape[0]
    nhid = w1.shape[1]
    nclass = w2.shape[1]

    tm = 512 if n % 512 == 0 else 128

    w1b = w1.astype(jnp.bfloat16)
    w2b = w2.astype(jnp.bfloat16)
    b1r = b1.astype(jnp.float32).reshape(1, nhid)
    b2r = b2.astype(jnp.float32).reshape(1, nclass)

    return _gcn_call(x, adj, w1b, b1r, w2b, b2r, tm=tm)
